# baseline (speedup 1.0000x reference)
import functools

import jax
import jax.numpy as jnp
from jax import lax
from jax.experimental import pallas as pl
from jax.experimental.pallas import tpu as pltpu
from jax.experimental.pallas import tpu_sc as plsc


_CHUNK = 128


@functools.lru_cache(maxsize=None)
def _make_gather(n, v, d):
    info = plsc.get_sparse_core_info()
    nc, ns = info.num_cores, info.num_subcores
    nw = nc * ns
    assert n % (nw * _CHUNK) == 0
    per_w = n // nw
    nchunks = per_w // _CHUNK

    mesh = plsc.VectorSubcoreMesh(core_axis_name="c", subcore_axis_name="s")

    nbuf = 2
    assert nchunks % nbuf == 0

    @functools.partial(
        pl.kernel,
        mesh=mesh,
        out_type=jax.ShapeDtypeStruct((n, d), jnp.float32),
        scratch_types=[
            pltpu.VMEM((per_w,), jnp.int32),
            pltpu.VMEM((nbuf, _CHUNK, d), jnp.float32),
            pltpu.SemaphoreType.DMA((nbuf,)),
            pltpu.SemaphoreType.DMA((nbuf,)),
        ],
    )
    def body(x_hbm, w_hbm, out_hbm, idx_v, rows_v, gsem, wsem):
        wid = lax.axis_index("s") * nc + lax.axis_index("c")
        base = wid * per_w

        pltpu.sync_copy(x_hbm.at[pl.ds(base, per_w)], idx_v)

        def start_gather(chunk, b):
            pltpu.async_copy(
                w_hbm.at[idx_v.at[pl.ds(chunk * _CHUNK, _CHUNK)]],
                rows_v.at[b],
                gsem.at[b],
            )

        def wait_gather(b):
            pltpu.make_async_copy(
                w_hbm.at[idx_v.at[pl.ds(0, _CHUNK)]],
                rows_v.at[b],
                gsem.at[b],
            ).wait()

        def start_write(chunk, b):
            pltpu.async_copy(
                rows_v.at[b],
                out_hbm.at[pl.ds(base + chunk * _CHUNK, _CHUNK)],
                wsem.at[b],
            )

        def wait_write(b):
            pltpu.make_async_copy(
                rows_v.at[b],
                out_hbm.at[pl.ds(base, _CHUNK)],
                wsem.at[b],
            ).wait()

        for b in range(nbuf):
            start_gather(b, b)

        def step(g, carry):
            for b in range(nbuf):
                chunk = g * nbuf + b
                wait_gather(b)
                start_write(chunk, b)
            for b in range(nbuf):
                nxt = (g + 1) * nbuf + b

                @pl.when(nxt < nchunks)
                def _():
                    wait_write(b)
                    start_gather(nxt, b)

            return carry

        lax.fori_loop(0, nchunks // nbuf, step, 0)

        for b in range(nbuf):
            wait_write(b)

    return body


def kernel(x, W):
    b, s = x.shape
    v, d = W.shape
    n = b * s
    out = _make_gather(n, v, d)(x.reshape(n), W)
    return out.reshape(b, s, d)


# speedup vs baseline: 1.0845x; 1.0845x over previous
import functools

import jax
import jax.numpy as jnp
from jax import lax
from jax.experimental import pallas as pl
from jax.experimental.pallas import tpu as pltpu
from jax.experimental.pallas import tpu_sc as plsc


_CHUNK = 128


@functools.lru_cache(maxsize=None)
def _make_gather(n, v, d):
    info = plsc.get_sparse_core_info()
    nc, ns = info.num_cores, info.num_subcores
    nw = nc * ns
    assert n % (nw * _CHUNK) == 0
    per_w = n // nw
    nchunks = per_w // _CHUNK

    mesh = plsc.VectorSubcoreMesh(core_axis_name="c", subcore_axis_name="s")

    gpb = 2
    group = gpb * _CHUNK
    ngroups = per_w // group
    assert per_w % group == 0 and ngroups % 2 == 0

    @functools.partial(
        pl.kernel,
        mesh=mesh,
        out_type=jax.ShapeDtypeStruct((n, d), jnp.float32),
        scratch_types=[
            pltpu.VMEM((per_w,), jnp.int32),
            pltpu.VMEM((2, group, d), jnp.float32),
            pltpu.SemaphoreType.DMA,
            pltpu.SemaphoreType.DMA,
        ],
    )
    def body(x_hbm, w_hbm, out_hbm, idx_v, rows_v, gsem0, gsem1):
        wid = lax.axis_index("s") * nc + lax.axis_index("c")
        base = wid * per_w
        gsems = (gsem0, gsem1)

        pltpu.sync_copy(x_hbm.at[pl.ds(base, per_w)], idx_v)

        def start_gather(grp, b):
            for k in range(gpb):
                pltpu.async_copy(
                    w_hbm.at[idx_v.at[pl.ds(grp * group + k * _CHUNK, _CHUNK)]],
                    rows_v.at[b].at[pl.ds(k * _CHUNK, _CHUNK)],
                    gsems[b],
                )

        start_gather(0, 0)
        start_gather(1, 1)

        def step(g, carry):
            for b in range(2):
                grp = g * 2 + b
                for k in range(gpb):
                    pltpu.make_async_copy(
                        w_hbm.at[idx_v.at[pl.ds(0, _CHUNK)]],
                        rows_v.at[b].at[pl.ds(k * _CHUNK, _CHUNK)],
                        gsems[b],
                    ).wait()
                pltpu.sync_copy(
                    rows_v.at[b],
                    out_hbm.at[pl.ds(base + grp * group, group)],
                )

                @pl.when(grp + 2 < ngroups)
                def _():
                    start_gather(grp + 2, b)

            return carry

        lax.fori_loop(0, ngroups // 2, step, 0)

    return body


def kernel(x, W):
    b, s = x.shape
    v, d = W.shape
    n = b * s
    out = _make_gather(n, v, d)(x.reshape(n), W)
    return out.reshape(b, s, d)
